# Initial kernel scaffold; baseline (speedup 1.0000x reference)
import jax
import jax.numpy as jnp
from jax.experimental import pallas as pl


def kernel(x_nchw, w1, w2):
    raise NotImplementedError("write your pallas kernel here")


# speedup vs baseline: 1.1755x; 1.1755x over previous
import functools

import jax
import jax.numpy as jnp
from jax.experimental import pallas as pl
from jax.experimental.pallas import tpu as pltpu


def _se_fused_body(x_ref, w1t_ref, w2t_ref, o_ref, *, inv_hw):
    x = x_ref[...]

    pooled = jnp.sum(x, axis=2, dtype=jnp.float32) * inv_hw

    hid = jnp.dot(pooled, w1t_ref[...], preferred_element_type=jnp.float32)
    hid = jnp.maximum(hid, 0.0)
    logits = jnp.dot(hid, w2t_ref[...], preferred_element_type=jnp.float32)
    gate = jax.nn.sigmoid(logits).astype(x.dtype)

    o_ref[...] = x * gate[:, :, None]


def _batch_tile(B, C, HW, itemsize):
    lanes = -(-HW // 128) * 128
    slab = C * lanes * itemsize
    budget = 44 << 20
    fit = max(int(budget // (4 * slab)), 1)
    bt = min(fit, max(B // 2, 1))
    while bt > 1 and B % bt:
        bt -= 1
    return bt


def kernel(x_nchw, w1, w2):
    B, C, H, W = x_nchw.shape
    hidden = w1.shape[0]
    HW = H * W
    dt = x_nchw.dtype
    itemsize = jnp.dtype(dt).itemsize

    x_flat = x_nchw.reshape(B, C, HW)

    bt = _batch_tile(B, C, HW, itemsize)
    grid = B // bt

    w1t = w1.T.astype(jnp.float32)
    w2t = w2.T.astype(jnp.float32)

    cost = pl.CostEstimate(
        flops=B * (2 * C * HW + 4 * C * hidden),
        transcendentals=B * C,
        bytes_accessed=2 * B * C * HW * itemsize,
    )

    out_flat = pl.pallas_call(
        functools.partial(_se_fused_body, inv_hw=1.0 / HW),
        out_shape=jax.ShapeDtypeStruct((B, C, HW), dt),
        grid=(grid,),
        in_specs=[
            pl.BlockSpec((bt, C, HW), lambda b: (b, 0, 0)),
            pl.BlockSpec((C, hidden), lambda b: (0, 0)),
            pl.BlockSpec((hidden, C), lambda b: (0, 0)),
        ],
        out_specs=pl.BlockSpec((bt, C, HW), lambda b: (b, 0, 0)),
        compiler_params=pltpu.CompilerParams(
            dimension_semantics=("parallel",),
            vmem_limit_bytes=60 << 20,
        ),
        cost_estimate=cost,
    )(x_flat, w1t, w2t)

    return out_flat.reshape(B, C, H, W)
